# baseline (speedup 1.0000x reference)
import jax
import jax.numpy as jnp
from jax.experimental import pallas as pl
from jax.experimental.pallas import tpu as pltpu

NCLS = 21
_VMEM_LIMIT = 48 * 1024 * 1024
_INTERPRET = False


def _cparams(sem):
    return pltpu.CompilerParams(
        dimension_semantics=sem,
        vmem_limit_bytes=_VMEM_LIMIT,
    )


def _conv_kernel_body(offsets, M, Cin, kk, relu, l2s, G):
    def body(*refs):
        refs = list(refs)
        x_ref, w_ref, b_ref = refs[:3]
        refs = refs[3:]
        s_ref = refs.pop(0) if l2s else None
        out_ref = refs.pop(0)
        f1_ref = refs.pop(0) if l2s else None
        sc = refs.pop(0) if kk > 1 else None

        def matmul():
            if kk == 1:
                return jnp.dot(x_ref[0, 0], w_ref[...],
                               preferred_element_type=jnp.float32)
            if G == 1:
                for j, o in enumerate(offsets):
                    sc[:, j * Cin:(j + 1) * Cin] = x_ref[0, 0, o:o + M, :]
            else:
                @pl.when(pl.program_id(2) == 0)
                def _():
                    for j, o in enumerate(offsets):
                        sc[:, j * Cin:(j + 1) * Cin] = x_ref[0, 0, o:o + M, :]
            return jnp.dot(sc[...], w_ref[...], preferred_element_type=jnp.float32)

        r = matmul() + b_ref[...]
        if relu:
            r = jnp.maximum(r, 0.0)
        if l2s:
            ss = jnp.sum(r * r, axis=1, keepdims=True)
            nrm = jnp.maximum(jnp.sqrt(ss), 1e-12)
            f1_ref[0, 0, 0:M] = r * (s_ref[...] / nrm)
        out_ref[0, 0, 0:M] = r

    return body


def conv2d(x, w, b, *, pad=1, dil=1, T=1, G=1, relu=True, l2_scale=None):
    N, H, W, Cin = x.shape
    Cout, _, k, _ = w.shape
    hal = (k - 1) * dil
    if pad:
        x = jnp.pad(x, ((0, 0), (pad, pad), (pad, pad), (0, 0)))
    Hp, Wp = H + 2 * pad, W + 2 * pad
    Hout, Wout = Hp - hal, Wp - hal
    assert Hout % T == 0 and Cout % G == 0, (Hout, T, Cout, G)
    TH = Hout // T
    CG = Cout // G
    if T > 1:
        xt = jnp.stack([x[:, t * TH:t * TH + TH + hal] for t in range(T)], axis=1)
    else:
        xt = x[:, None]
    LHW = (TH + hal) * Wp
    xt = xt.reshape(N, T, LHW, Cin)
    M = (TH - 1) * Wp + Wout
    THW = TH * Wp
    wmat = w.transpose(2, 3, 1, 0).reshape(k * k * Cin, Cout)
    bias = b.reshape(1, Cout)
    offsets = [(kh * dil) * Wp + kw * dil for kh in range(k) for kw in range(k)]

    in_specs = [
        pl.BlockSpec((1, 1, LHW, Cin), lambda n, t, g: (n, t, 0, 0)),
        pl.BlockSpec((k * k * Cin, CG), lambda n, t, g: (0, g)),
        pl.BlockSpec((1, CG), lambda n, t, g: (0, g)),
    ]
    out_spec = pl.BlockSpec((1, 1, THW, CG), lambda n, t, g: (n, t, 0, g))
    out_shapes = jax.ShapeDtypeStruct((N, T, THW, Cout), jnp.float32)
    l2s = l2_scale is not None
    if l2s:
        in_specs.append(pl.BlockSpec((1, CG), lambda n, t, g: (0, g)))
        out_spec = [out_spec, pl.BlockSpec((1, 1, THW, CG), lambda n, t, g: (n, t, 0, g))]
        out_shapes = [out_shapes, jax.ShapeDtypeStruct((N, T, THW, Cout), jnp.float32)]

    scratch = [pltpu.VMEM((M, k * k * Cin), jnp.float32)] if k > 1 else []
    args = (xt, wmat, bias) + ((l2_scale.reshape(1, Cout),) if l2s else ())
    out = pl.pallas_call(
        _conv_kernel_body(offsets, M, Cin, k, relu, l2s, G),
        grid=(N, T, G),
        in_specs=in_specs,
        out_specs=out_spec,
        out_shape=out_shapes,
        scratch_shapes=scratch,
        compiler_params=_cparams(("parallel", "parallel", "arbitrary")),
        name="conv",
        interpret=_INTERPRET,
    )(*args)
    if l2s:
        y, f1 = out
        return (y.reshape(N, Hout, Wp, Cout), f1.reshape(N, Hout, Wp, Cout))
    return out.reshape(N, Hout, Wp, Cout)


def conv_valid(x, w, b, *, pad=1, dil=1, T=1, G=1, relu=True):
    N, H, W, _ = x.shape
    k = w.shape[2]
    Wout = W + 2 * pad - (k - 1) * dil
    y = conv2d(x, w, b, pad=pad, dil=dil, T=T, G=G, relu=relu)
    return y[:, :, :Wout, :]


def _pool22_body(x_ref, o_ref):
    xx = x_ref[0]
    H2, W, C = xx.shape
    xr = xx.reshape(H2 // 2, 2, W // 2, 2, C)
    o_ref[0] = jnp.max(jnp.max(xr, axis=3), axis=1)


def maxpool22(x, T=1):
    N, H, W, C = x.shape
    Ho, Wo = H // 2, W // 2
    assert Ho % T == 0
    THo = Ho // T
    return pl.pallas_call(
        _pool22_body,
        grid=(N * T,),
        in_specs=[pl.BlockSpec((1, 2 * THo, W, C), lambda i: (i, 0, 0, 0))],
        out_specs=pl.BlockSpec((1, THo, Wo, C), lambda i: (i, 0, 0, 0)),
        out_shape=jax.ShapeDtypeStruct((N * T, THo, Wo, C), jnp.float32),
        compiler_params=_cparams(("parallel",)),
        name="pool22",
        interpret=_INTERPRET,
    )(x.reshape(N * T, 2 * THo, W, C)).reshape(N, Ho, Wo, C)


def _pool331_body(x_ref, o_ref):
    xx = x_ref[0]
    Hp, Wp, C = xx.shape
    H, W = Hp - 2, Wp - 2
    r = xx[0:H, 0:W]
    for dy in range(3):
        for dx in range(3):
            if dy == 0 and dx == 0:
                continue
            r = jnp.maximum(r, xx[dy:dy + H, dx:dx + W])
    o_ref[0] = r


def maxpool331(x):
    N, H, W, C = x.shape
    xp = jnp.pad(x, ((0, 0), (1, 1), (1, 1), (0, 0)), constant_values=-jnp.inf)
    return pl.pallas_call(
        _pool331_body,
        grid=(N,),
        in_specs=[pl.BlockSpec((1, H + 2, W + 2, C), lambda n: (n, 0, 0, 0))],
        out_specs=pl.BlockSpec((1, H, W, C), lambda n: (n, 0, 0, 0)),
        out_shape=jax.ShapeDtypeStruct((N, H, W, C), jnp.float32),
        compiler_params=_cparams(("parallel",)),
        name="pool331",
        interpret=_INTERPRET,
    )(xp)


def kernel(x, scale_weight, vgg, conv5fc, extras, cls_heads, bbox_heads):
    N = x.shape[0]
    h = x.transpose(0, 2, 3, 1)

    h = conv_valid(h, *vgg[0], T=10)
    h = conv_valid(h, *vgg[1], T=10)
    h = maxpool22(h, T=6)
    h = conv_valid(h, *vgg[2], T=6)
    h = conv_valid(h, *vgg[3], T=6)
    h = maxpool22(h, T=3)
    h = conv_valid(h, *vgg[4], T=3)
    h = conv_valid(h, *vgg[5], T=3)
    h = conv_valid(h, *vgg[6], T=3)
    h = jnp.pad(h, ((0, 0), (0, 1), (0, 1), (0, 0)), constant_values=-jnp.inf)
    h = maxpool22(h, T=2)
    h = conv_valid(h, *vgg[7], T=2)
    h = conv_valid(h, *vgg[8], T=2)
    c43, f1 = conv2d(h, *vgg[9], T=2, l2_scale=scale_weight)
    c43 = c43[:, :, :38, :]
    f1 = f1[:, :, :38, :]
    h = maxpool22(c43)
    h = conv_valid(h, *conv5fc[0])
    h = conv_valid(h, *conv5fc[1])
    h = conv_valid(h, *conv5fc[2])
    h = maxpool331(h)
    h = conv_valid(h, *conv5fc[3], pad=6, dil=6, G=4)
    f2 = conv_valid(h, *conv5fc[4], pad=0)
    h = conv_valid(f2, *extras[0], pad=0)
    h = conv_valid(h, *extras[1], pad=0)
    f3 = h[:, ::2, ::2, :]
    h = conv_valid(f3, *extras[2], pad=0)
    h = conv_valid(h, *extras[3], pad=1)
    f4 = h[:, ::2, ::2, :]
    h = conv_valid(f4, *extras[4], pad=0)
    f5 = conv_valid(h, *extras[5], pad=0)
    h = conv_valid(f5, *extras[6], pad=0)
    f6 = conv_valid(h, *extras[7], pad=0)

    feats = [f1, f2, f3, f4, f5, f6]
    head_T = [2, 1, 1, 1, 1, 1]
    cls_all, box_all = [], []
    for f, pc, pb, T in zip(feats, cls_heads, bbox_heads, head_T):
        wc, bc = pc
        wb, bb = pb
        A = wc.shape[0] // NCLS
        wcat = jnp.concatenate([wc, wb], axis=0)
        bcat = jnp.concatenate([bc, bb], axis=0)
        y = conv_valid(f, wcat, bcat, pad=1, T=T, relu=False)
        H, W = f.shape[1], f.shape[2]
        c = y[..., :A * NCLS].reshape(N, H * W * A, NCLS)
        bx = y[..., A * NCLS:].reshape(N, H * W * A, 4)
        cls_all.append(c)
        box_all.append(bx)
    cls_logits = jnp.concatenate(cls_all, axis=1)
    bbox_deltas = jnp.concatenate(box_all, axis=1)
    return jnp.concatenate([cls_logits, bbox_deltas], axis=-1)


# speedup vs baseline: 1.0041x; 1.0041x over previous
import jax
import jax.numpy as jnp
from jax.experimental import pallas as pl
from jax.experimental.pallas import tpu as pltpu

NCLS = 21
_VMEM_LIMIT = 48 * 1024 * 1024
_INTERPRET = False


def _cparams(sem):
    return pltpu.CompilerParams(
        dimension_semantics=sem,
        vmem_limit_bytes=_VMEM_LIMIT,
    )


def _conv_kernel_body(offsets, M, Cin, kk, relu, l2s, G):
    def body(*refs):
        refs = list(refs)
        x_ref, w_ref, b_ref = refs[:3]
        refs = refs[3:]
        s_ref = refs.pop(0) if l2s else None
        out_ref = refs.pop(0)
        f1_ref = refs.pop(0) if l2s else None
        sc = refs.pop(0) if kk > 1 else None

        def matmul():
            if kk == 1:
                return jnp.dot(x_ref[0, 0].astype(jnp.bfloat16), w_ref[...],
                               preferred_element_type=jnp.float32)
            if G == 1:
                for j, o in enumerate(offsets):
                    sc[:, j * Cin:(j + 1) * Cin] = \
                        x_ref[0, 0, o:o + M, :].astype(jnp.bfloat16)
            else:
                @pl.when(pl.program_id(2) == 0)
                def _():
                    for j, o in enumerate(offsets):
                        sc[:, j * Cin:(j + 1) * Cin] = \
                            x_ref[0, 0, o:o + M, :].astype(jnp.bfloat16)
            return jnp.dot(sc[...], w_ref[...], preferred_element_type=jnp.float32)

        r = matmul() + b_ref[...]
        if relu:
            r = jnp.maximum(r, 0.0)
        if l2s:
            ss = jnp.sum(r * r, axis=1, keepdims=True)
            nrm = jnp.maximum(jnp.sqrt(ss), 1e-12)
            f1_ref[0, 0, 0:M] = r * (s_ref[...] / nrm)
        out_ref[0, 0, 0:M] = r

    return body


def conv2d(x, w, b, *, pad=1, dil=1, T=1, G=1, relu=True, l2_scale=None):
    N, H, W, Cin = x.shape
    Cout, _, k, _ = w.shape
    hal = (k - 1) * dil
    if pad:
        x = jnp.pad(x, ((0, 0), (pad, pad), (pad, pad), (0, 0)))
    Hp, Wp = H + 2 * pad, W + 2 * pad
    Hout, Wout = Hp - hal, Wp - hal
    assert Hout % T == 0 and Cout % G == 0, (Hout, T, Cout, G)
    TH = Hout // T
    CG = Cout // G
    if T > 1:
        xt = jnp.stack([x[:, t * TH:t * TH + TH + hal] for t in range(T)], axis=1)
    else:
        xt = x[:, None]
    LHW = (TH + hal) * Wp
    xt = xt.reshape(N, T, LHW, Cin)
    M = (TH - 1) * Wp + Wout
    THW = TH * Wp
    wmat = w.transpose(2, 3, 1, 0).reshape(k * k * Cin, Cout).astype(jnp.bfloat16)
    bias = b.reshape(1, Cout)
    offsets = [(kh * dil) * Wp + kw * dil for kh in range(k) for kw in range(k)]

    in_specs = [
        pl.BlockSpec((1, 1, LHW, Cin), lambda n, t, g: (n, t, 0, 0)),
        pl.BlockSpec((k * k * Cin, CG), lambda n, t, g: (0, g)),
        pl.BlockSpec((1, CG), lambda n, t, g: (0, g)),
    ]
    out_spec = pl.BlockSpec((1, 1, THW, CG), lambda n, t, g: (n, t, 0, g))
    out_shapes = jax.ShapeDtypeStruct((N, T, THW, Cout), jnp.float32)
    l2s = l2_scale is not None
    if l2s:
        in_specs.append(pl.BlockSpec((1, CG), lambda n, t, g: (0, g)))
        out_spec = [out_spec, pl.BlockSpec((1, 1, THW, CG), lambda n, t, g: (n, t, 0, g))]
        out_shapes = [out_shapes, jax.ShapeDtypeStruct((N, T, THW, Cout), jnp.float32)]

    scratch = [pltpu.VMEM((M, k * k * Cin), jnp.bfloat16)] if k > 1 else []
    args = (xt, wmat, bias) + ((l2_scale.reshape(1, Cout),) if l2s else ())
    out = pl.pallas_call(
        _conv_kernel_body(offsets, M, Cin, k, relu, l2s, G),
        grid=(N, T, G),
        in_specs=in_specs,
        out_specs=out_spec,
        out_shape=out_shapes,
        scratch_shapes=scratch,
        compiler_params=_cparams(("parallel", "parallel", "arbitrary")),
        name="conv",
        interpret=_INTERPRET,
    )(*args)
    if l2s:
        y, f1 = out
        return (y.reshape(N, Hout, Wp, Cout), f1.reshape(N, Hout, Wp, Cout))
    return out.reshape(N, Hout, Wp, Cout)


def conv_valid(x, w, b, *, pad=1, dil=1, T=1, G=1, relu=True):
    N, H, W, _ = x.shape
    k = w.shape[2]
    Wout = W + 2 * pad - (k - 1) * dil
    y = conv2d(x, w, b, pad=pad, dil=dil, T=T, G=G, relu=relu)
    return y[:, :, :Wout, :]


def _pool22_body(x_ref, o_ref):
    xx = x_ref[0]
    H2, W, C = xx.shape
    xr = xx.reshape(H2 // 2, 2, W // 2, 2, C)
    o_ref[0] = jnp.max(jnp.max(xr, axis=3), axis=1)


def maxpool22(x, T=1):
    N, H, W, C = x.shape
    Ho, Wo = H // 2, W // 2
    assert Ho % T == 0
    THo = Ho // T
    return pl.pallas_call(
        _pool22_body,
        grid=(N * T,),
        in_specs=[pl.BlockSpec((1, 2 * THo, W, C), lambda i: (i, 0, 0, 0))],
        out_specs=pl.BlockSpec((1, THo, Wo, C), lambda i: (i, 0, 0, 0)),
        out_shape=jax.ShapeDtypeStruct((N * T, THo, Wo, C), jnp.float32),
        compiler_params=_cparams(("parallel",)),
        name="pool22",
        interpret=_INTERPRET,
    )(x.reshape(N * T, 2 * THo, W, C)).reshape(N, Ho, Wo, C)


def _pool331_body(x_ref, o_ref):
    xx = x_ref[0]
    Hp, Wp, C = xx.shape
    H, W = Hp - 2, Wp - 2
    r = xx[0:H, 0:W]
    for dy in range(3):
        for dx in range(3):
            if dy == 0 and dx == 0:
                continue
            r = jnp.maximum(r, xx[dy:dy + H, dx:dx + W])
    o_ref[0] = r


def maxpool331(x):
    N, H, W, C = x.shape
    xp = jnp.pad(x, ((0, 0), (1, 1), (1, 1), (0, 0)), constant_values=-jnp.inf)
    return pl.pallas_call(
        _pool331_body,
        grid=(N,),
        in_specs=[pl.BlockSpec((1, H + 2, W + 2, C), lambda n: (n, 0, 0, 0))],
        out_specs=pl.BlockSpec((1, H, W, C), lambda n: (n, 0, 0, 0)),
        out_shape=jax.ShapeDtypeStruct((N, H, W, C), jnp.float32),
        compiler_params=_cparams(("parallel",)),
        name="pool331",
        interpret=_INTERPRET,
    )(xp)


def kernel(x, scale_weight, vgg, conv5fc, extras, cls_heads, bbox_heads):
    N = x.shape[0]
    h = x.transpose(0, 2, 3, 1)

    h = conv_valid(h, *vgg[0], T=10)
    h = conv_valid(h, *vgg[1], T=10)
    h = maxpool22(h, T=6)
    h = conv_valid(h, *vgg[2], T=6)
    h = conv_valid(h, *vgg[3], T=6)
    h = maxpool22(h, T=3)
    h = conv_valid(h, *vgg[4], T=3)
    h = conv_valid(h, *vgg[5], T=3)
    h = conv_valid(h, *vgg[6], T=3)
    h = jnp.pad(h, ((0, 0), (0, 1), (0, 1), (0, 0)), constant_values=-jnp.inf)
    h = maxpool22(h, T=2)
    h = conv_valid(h, *vgg[7], T=2)
    h = conv_valid(h, *vgg[8], T=2)
    c43, f1 = conv2d(h, *vgg[9], T=2, l2_scale=scale_weight)
    c43 = c43[:, :, :38, :]
    f1 = f1[:, :, :38, :]
    h = maxpool22(c43)
    h = conv_valid(h, *conv5fc[0])
    h = conv_valid(h, *conv5fc[1])
    h = conv_valid(h, *conv5fc[2])
    h = maxpool331(h)
    h = conv_valid(h, *conv5fc[3], pad=6, dil=6, G=4)
    f2 = conv_valid(h, *conv5fc[4], pad=0)
    h = conv_valid(f2, *extras[0], pad=0)
    h = conv_valid(h, *extras[1], pad=0)
    f3 = h[:, ::2, ::2, :]
    h = conv_valid(f3, *extras[2], pad=0)
    h = conv_valid(h, *extras[3], pad=1)
    f4 = h[:, ::2, ::2, :]
    h = conv_valid(f4, *extras[4], pad=0)
    f5 = conv_valid(h, *extras[5], pad=0)
    h = conv_valid(f5, *extras[6], pad=0)
    f6 = conv_valid(h, *extras[7], pad=0)

    feats = [f1, f2, f3, f4, f5, f6]
    head_T = [2, 1, 1, 1, 1, 1]
    cls_all, box_all = [], []
    for f, pc, pb, T in zip(feats, cls_heads, bbox_heads, head_T):
        wc, bc = pc
        wb, bb = pb
        A = wc.shape[0] // NCLS
        wcat = jnp.concatenate([wc, wb], axis=0)
        bcat = jnp.concatenate([bc, bb], axis=0)
        y = conv_valid(f, wcat, bcat, pad=1, T=T, relu=False)
        H, W = f.shape[1], f.shape[2]
        c = y[..., :A * NCLS].reshape(N, H * W * A, NCLS)
        bx = y[..., A * NCLS:].reshape(N, H * W * A, 4)
        cls_all.append(c)
        box_all.append(bx)
    cls_logits = jnp.concatenate(cls_all, axis=1)
    bbox_deltas = jnp.concatenate(box_all, axis=1)
    return jnp.concatenate([cls_logits, bbox_deltas], axis=-1)


# speedup vs baseline: 1.0057x; 1.0016x over previous
import jax
import jax.numpy as jnp
from jax.experimental import pallas as pl
from jax.experimental.pallas import tpu as pltpu

NCLS = 21
_VMEM_LIMIT = 48 * 1024 * 1024
_INTERPRET = False


def _cparams(sem):
    return pltpu.CompilerParams(
        dimension_semantics=sem,
        vmem_limit_bytes=_VMEM_LIMIT,
    )


def _conv_kernel_body(offsets, M, Cin, kk, relu, l2s, G):
    def body(*refs):
        refs = list(refs)
        x_ref, w_ref, b_ref = refs[:3]
        refs = refs[3:]
        s_ref = refs.pop(0) if l2s else None
        out_ref = refs.pop(0)
        f1_ref = refs.pop(0) if l2s else None
        sc = refs.pop(0) if kk > 1 else None

        def matmul():
            if kk == 1:
                return jnp.dot(x_ref[0, 0].astype(jnp.bfloat16), w_ref[...],
                               preferred_element_type=jnp.float32)
            if G == 1:
                for j, o in enumerate(offsets):
                    sc[:, j * Cin:(j + 1) * Cin] = \
                        x_ref[0, 0, o:o + M, :].astype(jnp.bfloat16)
            else:
                @pl.when(pl.program_id(2) == 0)
                def _():
                    for j, o in enumerate(offsets):
                        sc[:, j * Cin:(j + 1) * Cin] = \
                            x_ref[0, 0, o:o + M, :].astype(jnp.bfloat16)
            return jnp.dot(sc[...], w_ref[...], preferred_element_type=jnp.float32)

        r = matmul() + b_ref[...]
        if relu:
            r = jnp.maximum(r, 0.0)
        if l2s:
            ss = jnp.sum(r * r, axis=1, keepdims=True)
            nrm = jnp.maximum(jnp.sqrt(ss), 1e-12)
            f1_ref[0, 0, 0:M] = r * (s_ref[...] / nrm)
        out_ref[0, 0, 0:M] = r

    return body


def conv2d(x, w, b, *, pad=1, dil=1, T=1, G=1, relu=True, l2_scale=None):
    N, H, W, Cin = x.shape
    Cout, _, k, _ = w.shape
    hal = (k - 1) * dil
    if pad:
        x = jnp.pad(x, ((0, 0), (pad, pad), (pad, pad), (0, 0)))
    Hp, Wp = H + 2 * pad, W + 2 * pad
    Hout, Wout = Hp - hal, Wp - hal
    assert Hout % T == 0 and Cout % G == 0, (Hout, T, Cout, G)
    TH = Hout // T
    CG = Cout // G
    if T > 1:
        xt = jnp.stack([x[:, t * TH:t * TH + TH + hal] for t in range(T)], axis=1)
    else:
        xt = x[:, None]
    LHW = (TH + hal) * Wp
    xt = xt.reshape(N, T, LHW, Cin)
    M = (TH - 1) * Wp + Wout
    THW = TH * Wp
    wmat = w.transpose(2, 3, 1, 0).reshape(k * k * Cin, Cout).astype(jnp.bfloat16)
    bias = b.reshape(1, Cout)
    offsets = [(kh * dil) * Wp + kw * dil for kh in range(k) for kw in range(k)]

    in_specs = [
        pl.BlockSpec((1, 1, LHW, Cin), lambda n, t, g: (n, t, 0, 0)),
        pl.BlockSpec((k * k * Cin, CG), lambda n, t, g: (0, g)),
        pl.BlockSpec((1, CG), lambda n, t, g: (0, g)),
    ]
    out_spec = pl.BlockSpec((1, 1, THW, CG), lambda n, t, g: (n, t, 0, g))
    out_shapes = jax.ShapeDtypeStruct((N, T, THW, Cout), jnp.float32)
    l2s = l2_scale is not None
    if l2s:
        in_specs.append(pl.BlockSpec((1, CG), lambda n, t, g: (0, g)))
        out_spec = [out_spec, pl.BlockSpec((1, 1, THW, CG), lambda n, t, g: (n, t, 0, g))]
        out_shapes = [out_shapes, jax.ShapeDtypeStruct((N, T, THW, Cout), jnp.float32)]

    scratch = [pltpu.VMEM((M, k * k * Cin), jnp.bfloat16)] if k > 1 else []
    args = (xt, wmat, bias) + ((l2_scale.reshape(1, Cout),) if l2s else ())
    out = pl.pallas_call(
        _conv_kernel_body(offsets, M, Cin, k, relu, l2s, G),
        grid=(N, T, G),
        in_specs=in_specs,
        out_specs=out_spec,
        out_shape=out_shapes,
        scratch_shapes=scratch,
        compiler_params=_cparams(("parallel", "parallel", "arbitrary")),
        name="conv",
        interpret=_INTERPRET,
    )(*args)
    if l2s:
        y, f1 = out
        return (y.reshape(N, Hout, Wp, Cout), f1.reshape(N, Hout, Wp, Cout))
    return out.reshape(N, Hout, Wp, Cout)


def conv_valid(x, w, b, *, pad=1, dil=1, T=1, G=1, relu=True):
    N, H, W, _ = x.shape
    k = w.shape[2]
    Wout = W + 2 * pad - (k - 1) * dil
    y = conv2d(x, w, b, pad=pad, dil=dil, T=T, G=G, relu=relu)
    return y[:, :, :Wout, :]


def _pool22_body(C):
    def body(x_ref, o_ref):
        v = x_ref[0]
        a = jnp.maximum(v[:, :, :C], v[:, :, C:])
        H2, Wo, _ = a.shape
        b = a.reshape(H2 // 2, 2, Wo, C)
        o_ref[0] = jnp.max(b, axis=1)
    return body


def maxpool22(x, T=1):
    N, H, W, C = x.shape
    Ho, Wo = H // 2, W // 2
    assert Ho % T == 0
    THo = Ho // T
    xv = x.reshape(N * T, 2 * THo, Wo, 2 * C)
    return pl.pallas_call(
        _pool22_body(C),
        grid=(N * T,),
        in_specs=[pl.BlockSpec((1, 2 * THo, Wo, 2 * C), lambda i: (i, 0, 0, 0))],
        out_specs=pl.BlockSpec((1, THo, Wo, C), lambda i: (i, 0, 0, 0)),
        out_shape=jax.ShapeDtypeStruct((N * T, THo, Wo, C), jnp.float32),
        compiler_params=_cparams(("parallel",)),
        name="pool22",
        interpret=_INTERPRET,
    )(xv).reshape(N, Ho, Wo, C)


def _pool331_body(x_ref, o_ref):
    xx = x_ref[0]
    Hp, Wp, C = xx.shape
    H, W = Hp - 2, Wp - 2
    r = xx[0:H, 0:W]
    for dy in range(3):
        for dx in range(3):
            if dy == 0 and dx == 0:
                continue
            r = jnp.maximum(r, xx[dy:dy + H, dx:dx + W])
    o_ref[0] = r


def maxpool331(x):
    N, H, W, C = x.shape
    xp = jnp.pad(x, ((0, 0), (1, 1), (1, 1), (0, 0)), constant_values=-jnp.inf)
    return pl.pallas_call(
        _pool331_body,
        grid=(N,),
        in_specs=[pl.BlockSpec((1, H + 2, W + 2, C), lambda n: (n, 0, 0, 0))],
        out_specs=pl.BlockSpec((1, H, W, C), lambda n: (n, 0, 0, 0)),
        out_shape=jax.ShapeDtypeStruct((N, H, W, C), jnp.float32),
        compiler_params=_cparams(("parallel",)),
        name="pool331",
        interpret=_INTERPRET,
    )(xp)


def kernel(x, scale_weight, vgg, conv5fc, extras, cls_heads, bbox_heads):
    N = x.shape[0]
    h = x.transpose(0, 2, 3, 1)

    h = conv_valid(h, *vgg[0], T=10)
    h = conv_valid(h, *vgg[1], T=10)
    h = maxpool22(h, T=6)
    h = conv_valid(h, *vgg[2], T=6)
    h = conv_valid(h, *vgg[3], T=6)
    h = maxpool22(h, T=3)
    h = conv_valid(h, *vgg[4], T=3)
    h = conv_valid(h, *vgg[5], T=3)
    h = conv_valid(h, *vgg[6], T=3)
    h = jnp.pad(h, ((0, 0), (0, 1), (0, 1), (0, 0)), constant_values=-jnp.inf)
    h = maxpool22(h, T=2)
    h = conv_valid(h, *vgg[7], T=2)
    h = conv_valid(h, *vgg[8], T=2)
    c43, f1 = conv2d(h, *vgg[9], T=2, l2_scale=scale_weight)
    c43 = c43[:, :, :38, :]
    f1 = f1[:, :, :38, :]
    h = maxpool22(c43)
    h = conv_valid(h, *conv5fc[0])
    h = conv_valid(h, *conv5fc[1])
    h = conv_valid(h, *conv5fc[2])
    h = maxpool331(h)
    h = conv_valid(h, *conv5fc[3], pad=6, dil=6, G=4)
    f2 = conv_valid(h, *conv5fc[4], pad=0)
    h = conv_valid(f2, *extras[0], pad=0)
    h = conv_valid(h, *extras[1], pad=0)
    f3 = h[:, ::2, ::2, :]
    h = conv_valid(f3, *extras[2], pad=0)
    h = conv_valid(h, *extras[3], pad=1)
    f4 = h[:, ::2, ::2, :]
    h = conv_valid(f4, *extras[4], pad=0)
    f5 = conv_valid(h, *extras[5], pad=0)
    h = conv_valid(f5, *extras[6], pad=0)
    f6 = conv_valid(h, *extras[7], pad=0)

    feats = [f1, f2, f3, f4, f5, f6]
    head_T = [2, 1, 1, 1, 1, 1]
    cls_all, box_all = [], []
    for f, pc, pb, T in zip(feats, cls_heads, bbox_heads, head_T):
        wc, bc = pc
        wb, bb = pb
        A = wc.shape[0] // NCLS
        wcat = jnp.concatenate([wc, wb], axis=0)
        bcat = jnp.concatenate([bc, bb], axis=0)
        y = conv_valid(f, wcat, bcat, pad=1, T=T, relu=False)
        H, W = f.shape[1], f.shape[2]
        c = y[..., :A * NCLS].reshape(N, H * W * A, NCLS)
        bx = y[..., A * NCLS:].reshape(N, H * W * A, 4)
        cls_all.append(c)
        box_all.append(bx)
    cls_logits = jnp.concatenate(cls_all, axis=1)
    bbox_deltas = jnp.concatenate(box_all, axis=1)
    return jnp.concatenate([cls_logits, bbox_deltas], axis=-1)


# speedup vs baseline: 1.0729x; 1.0669x over previous
import jax
import jax.numpy as jnp
from jax.experimental import pallas as pl
from jax.experimental.pallas import tpu as pltpu

NCLS = 21
_VMEM_LIMIT = 48 * 1024 * 1024
_INTERPRET = False


def _cparams(sem):
    return pltpu.CompilerParams(
        dimension_semantics=sem,
        vmem_limit_bytes=_VMEM_LIMIT,
    )


def _conv_kernel_body(offsets, M, Cin, kk, relu, l2s, G):
    def body(*refs):
        refs = list(refs)
        x_ref, w_ref, b_ref = refs[:3]
        refs = refs[3:]
        s_ref = refs.pop(0) if l2s else None
        out_ref = refs.pop(0)
        f1_ref = refs.pop(0) if l2s else None
        sc = refs.pop(0) if kk > 1 else None

        def matmul():
            if kk == 1:
                return jnp.dot(x_ref[0, 0].astype(jnp.bfloat16), w_ref[...],
                               preferred_element_type=jnp.float32)
            if G == 1:
                for j, o in enumerate(offsets):
                    sc[:, j * Cin:(j + 1) * Cin] = \
                        x_ref[0, 0, o:o + M, :].astype(jnp.bfloat16)
            else:
                @pl.when(pl.program_id(2) == 0)
                def _():
                    for j, o in enumerate(offsets):
                        sc[:, j * Cin:(j + 1) * Cin] = \
                            x_ref[0, 0, o:o + M, :].astype(jnp.bfloat16)
            return jnp.dot(sc[...], w_ref[...], preferred_element_type=jnp.float32)

        r = matmul() + b_ref[...]
        if relu:
            r = jnp.maximum(r, 0.0)
        if l2s:
            ss = jnp.sum(r * r, axis=1, keepdims=True)
            nrm = jnp.maximum(jnp.sqrt(ss), 1e-12)
            f1_ref[0, 0, 0:M] = r * (s_ref[...] / nrm)
        out_ref[0, 0, 0:M] = r

    return body


def conv2d(x, w, b, *, pad=1, dil=1, T=1, G=1, relu=True, l2_scale=None):
    N, H, W, Cin = x.shape
    Cout, _, k, _ = w.shape
    hal = (k - 1) * dil
    if pad:
        x = jnp.pad(x, ((0, 0), (pad, pad), (pad, pad), (0, 0)))
    Hp, Wp = H + 2 * pad, W + 2 * pad
    Hout, Wout = Hp - hal, Wp - hal
    assert Hout % T == 0 and Cout % G == 0, (Hout, T, Cout, G)
    TH = Hout // T
    CG = Cout // G
    if T > 1:
        xt = jnp.stack([x[:, t * TH:t * TH + TH + hal] for t in range(T)], axis=1)
    else:
        xt = x[:, None]
    LHW = (TH + hal) * Wp
    xt = xt.reshape(N, T, LHW, Cin)
    M = (TH - 1) * Wp + Wout
    THW = TH * Wp
    wmat = w.transpose(2, 3, 1, 0).reshape(k * k * Cin, Cout).astype(jnp.bfloat16)
    bias = b.reshape(1, Cout)
    offsets = [(kh * dil) * Wp + kw * dil for kh in range(k) for kw in range(k)]

    in_specs = [
        pl.BlockSpec((1, 1, LHW, Cin), lambda n, t, g: (n, t, 0, 0)),
        pl.BlockSpec((k * k * Cin, CG), lambda n, t, g: (0, g)),
        pl.BlockSpec((1, CG), lambda n, t, g: (0, g)),
    ]
    out_spec = pl.BlockSpec((1, 1, THW, CG), lambda n, t, g: (n, t, 0, g))
    out_shapes = jax.ShapeDtypeStruct((N, T, THW, Cout), jnp.float32)
    l2s = l2_scale is not None
    if l2s:
        in_specs.append(pl.BlockSpec((1, CG), lambda n, t, g: (0, g)))
        out_spec = [out_spec, pl.BlockSpec((1, 1, THW, CG), lambda n, t, g: (n, t, 0, g))]
        out_shapes = [out_shapes, jax.ShapeDtypeStruct((N, T, THW, Cout), jnp.float32)]

    scratch = [pltpu.VMEM((M, k * k * Cin), jnp.bfloat16)] if k > 1 else []
    args = (xt, wmat, bias) + ((l2_scale.reshape(1, Cout),) if l2s else ())
    out = pl.pallas_call(
        _conv_kernel_body(offsets, M, Cin, k, relu, l2s, G),
        grid=(N, T, G),
        in_specs=in_specs,
        out_specs=out_spec,
        out_shape=out_shapes,
        scratch_shapes=scratch,
        compiler_params=_cparams(("parallel", "parallel", "arbitrary")),
        name="conv",
        interpret=_INTERPRET,
    )(*args)
    if l2s:
        y, f1 = out
        return (y.reshape(N, Hout, Wp, Cout), f1.reshape(N, Hout, Wp, Cout))
    return out.reshape(N, Hout, Wp, Cout)


def _first_conv_body(M, bf16w):
    def body(x_ref, w_ref, b_ref, out_ref):
        r = jnp.dot(x_ref[0, 0].astype(jnp.bfloat16), w_ref[...],
                    preferred_element_type=jnp.float32)
        r = jnp.maximum(r + b_ref[...], 0.0)
        out_ref[0, 0, 0:M] = r
    return body


def first_conv(x, w, b):
    N, H, W, _ = x.shape
    Cout = w.shape[0]
    xp = jnp.pad(x, ((0, 0), (1, 1), (1, 1), (0, 0)))
    Hp = Wp = H + 2
    xpf = xp.reshape(N, Hp * Wp, 3)
    T = 10
    TH = H // T
    THW = TH * Wp
    M = (TH - 1) * Wp + W
    Mfull = (H - 1) * Wp + W
    offsets = [kh * Wp + kw for kh in range(3) for kw in range(3)]
    xc = jnp.concatenate([xpf[:, o:o + Mfull, :] for o in offsets], axis=2)
    xt = jnp.stack([xc[:, t * THW:t * THW + M] for t in range(T)], axis=1)
    wmat = w.transpose(2, 3, 1, 0).reshape(27, Cout).astype(jnp.bfloat16)
    out = pl.pallas_call(
        _first_conv_body(M, wmat),
        grid=(N, T),
        in_specs=[
            pl.BlockSpec((1, 1, M, 27), lambda n, t: (n, t, 0, 0)),
            pl.BlockSpec((27, Cout), lambda n, t: (0, 0)),
            pl.BlockSpec((1, Cout), lambda n, t: (0, 0)),
        ],
        out_specs=pl.BlockSpec((1, 1, THW, Cout), lambda n, t: (n, t, 0, 0)),
        out_shape=jax.ShapeDtypeStruct((N, T, THW, Cout), jnp.float32),
        compiler_params=_cparams(("parallel", "parallel")),
        name="conv1_1",
        interpret=_INTERPRET,
    )(xt, wmat, b.reshape(1, Cout))
    return out.reshape(N, H, Wp, Cout)[:, :, :W, :]


def conv_valid(x, w, b, *, pad=1, dil=1, T=1, G=1, relu=True):
    N, H, W, _ = x.shape
    k = w.shape[2]
    Wout = W + 2 * pad - (k - 1) * dil
    y = conv2d(x, w, b, pad=pad, dil=dil, T=T, G=G, relu=relu)
    return y[:, :, :Wout, :]


def _pool22_body(C):
    def body(x_ref, o_ref):
        v = x_ref[0]
        a = jnp.maximum(v[:, :, :C], v[:, :, C:])
        H2, Wo, _ = a.shape
        b = a.reshape(H2 // 2, 2, Wo, C)
        o_ref[0] = jnp.max(b, axis=1)
    return body


def maxpool22(x, T=1):
    N, H, W, C = x.shape
    Ho, Wo = H // 2, W // 2
    assert Ho % T == 0
    THo = Ho // T
    xv = x.reshape(N * T, 2 * THo, Wo, 2 * C)
    return pl.pallas_call(
        _pool22_body(C),
        grid=(N * T,),
        in_specs=[pl.BlockSpec((1, 2 * THo, Wo, 2 * C), lambda i: (i, 0, 0, 0))],
        out_specs=pl.BlockSpec((1, THo, Wo, C), lambda i: (i, 0, 0, 0)),
        out_shape=jax.ShapeDtypeStruct((N * T, THo, Wo, C), jnp.float32),
        compiler_params=_cparams(("parallel",)),
        name="pool22",
        interpret=_INTERPRET,
    )(xv).reshape(N, Ho, Wo, C)


def _pool331_body(x_ref, o_ref):
    xx = x_ref[0]
    Hp, Wp, C = xx.shape
    H, W = Hp - 2, Wp - 2
    r = xx[0:H, 0:W]
    for dy in range(3):
        for dx in range(3):
            if dy == 0 and dx == 0:
                continue
            r = jnp.maximum(r, xx[dy:dy + H, dx:dx + W])
    o_ref[0] = r


def maxpool331(x):
    N, H, W, C = x.shape
    xp = jnp.pad(x, ((0, 0), (1, 1), (1, 1), (0, 0)), constant_values=-jnp.inf)
    return pl.pallas_call(
        _pool331_body,
        grid=(N,),
        in_specs=[pl.BlockSpec((1, H + 2, W + 2, C), lambda n: (n, 0, 0, 0))],
        out_specs=pl.BlockSpec((1, H, W, C), lambda n: (n, 0, 0, 0)),
        out_shape=jax.ShapeDtypeStruct((N, H, W, C), jnp.float32),
        compiler_params=_cparams(("parallel",)),
        name="pool331",
        interpret=_INTERPRET,
    )(xp)


def kernel(x, scale_weight, vgg, conv5fc, extras, cls_heads, bbox_heads):
    N = x.shape[0]
    h = x.transpose(0, 2, 3, 1)

    h = first_conv(h, *vgg[0])
    h = conv_valid(h, *vgg[1], T=10)
    h = maxpool22(h, T=6)
    h = conv_valid(h, *vgg[2], T=6)
    h = conv_valid(h, *vgg[3], T=6)
    h = maxpool22(h, T=3)
    h = conv_valid(h, *vgg[4], T=3)
    h = conv_valid(h, *vgg[5], T=3)
    h = conv_valid(h, *vgg[6], T=3)
    h = jnp.pad(h, ((0, 0), (0, 1), (0, 1), (0, 0)), constant_values=-jnp.inf)
    h = maxpool22(h, T=2)
    h = conv_valid(h, *vgg[7], T=2)
    h = conv_valid(h, *vgg[8], T=2)
    c43, f1 = conv2d(h, *vgg[9], T=2, l2_scale=scale_weight)
    c43 = c43[:, :, :38, :]
    f1 = f1[:, :, :38, :]
    h = maxpool22(c43)
    h = conv_valid(h, *conv5fc[0])
    h = conv_valid(h, *conv5fc[1])
    h = conv_valid(h, *conv5fc[2])
    h = maxpool331(h)
    h = conv_valid(h, *conv5fc[3], pad=6, dil=6, G=4)
    f2 = conv_valid(h, *conv5fc[4], pad=0)
    h = conv_valid(f2, *extras[0], pad=0)
    h = conv_valid(h, *extras[1], pad=0)
    f3 = h[:, ::2, ::2, :]
    h = conv_valid(f3, *extras[2], pad=0)
    h = conv_valid(h, *extras[3], pad=1)
    f4 = h[:, ::2, ::2, :]
    h = conv_valid(f4, *extras[4], pad=0)
    f5 = conv_valid(h, *extras[5], pad=0)
    h = conv_valid(f5, *extras[6], pad=0)
    f6 = conv_valid(h, *extras[7], pad=0)

    feats = [f1, f2, f3, f4, f5, f6]
    head_T = [2, 1, 1, 1, 1, 1]
    cls_all, box_all = [], []
    for f, pc, pb, T in zip(feats, cls_heads, bbox_heads, head_T):
        wc, bc = pc
        wb, bb = pb
        A = wc.shape[0] // NCLS
        wcat = jnp.concatenate([wc, wb], axis=0)
        bcat = jnp.concatenate([bc, bb], axis=0)
        y = conv_valid(f, wcat, bcat, pad=1, T=T, relu=False)
        H, W = f.shape[1], f.shape[2]
        c = y[..., :A * NCLS].reshape(N, H * W * A, NCLS)
        bx = y[..., A * NCLS:].reshape(N, H * W * A, 4)
        cls_all.append(c)
        box_all.append(bx)
    cls_logits = jnp.concatenate(cls_all, axis=1)
    bbox_deltas = jnp.concatenate(box_all, axis=1)
    return jnp.concatenate([cls_logits, bbox_deltas], axis=-1)
